# baseline (speedup 1.0000x reference)
import functools

import jax
import jax.numpy as jnp
from jax import lax
from jax.experimental import pallas as pl
from jax.experimental.pallas import tpu as pltpu
from jax.experimental.pallas import tpu_sc as plsc

BATCH = 16384
EMB_K = 64
NROWS = 1000000
LANEB = 128
_LAST = (NROWS - 1) // LANEB
_CMAX = _LAST - 1
_TAIL = _LAST * LANEB

_info = plsc.get_sparse_core_info()
_NC, _NS = _info.num_cores, _info.num_subcores
_NW = _NC * _NS
NCHUNK = 2
CHUNK = BATCH // NCHUNK
_BPW = CHUNK // _NW
_HALF = _BPW // 2
_DEPTH = 8


def _extract_col(zb, l, dst, b):
    lvec = jnp.full((16,), l, jnp.int32)
    bvec = jnp.full((16,), b, jnp.int32)
    for g in range(EMB_K // 16):
        rows = lax.iota(jnp.int32, 16) + (g * 16)
        vals = plsc.load_gather(zb, [rows, lvec])
        plsc.store_scatter(dst, [rows, bvec], vals)


def _sc_gather_body(xt_hbm, w_t, h_t, zu_out, zv_out,
                    uidx_v, vidx_v, bufs, zb_v, sems):
    wid = lax.axis_index("s") * _NC + lax.axis_index("c")
    base = wid * _BPW
    pltpu.sync_copy(xt_hbm.at[0, pl.ds(base, _BPW)], uidx_v.at[pl.ds(0, _BPW)])
    pltpu.sync_copy(xt_hbm.at[1, pl.ds(base, _BPW)], vidx_v.at[pl.ds(0, _BPW)])

    def read_idx(idx_v, b):
        return idx_v[pl.ds(b, 16)][0]

    def fire(idx_v, table, buf, sem, b):
        i = read_idx(idx_v, b)
        c = jnp.minimum(lax.shift_right_logical(i, 7), _CMAX)
        o = pl.multiple_of(c * LANEB, LANEB)
        pltpu.make_async_copy(table.at[:, pl.ds(o, LANEB)], buf, sem).start()

    def take(idx_v, buf, sem, b, col):
        pltpu.make_async_copy(w_t.at[:, pl.ds(0, LANEB)], buf, sem).wait()
        i = read_idx(idx_v, b)
        c = jnp.minimum(lax.shift_right_logical(i, 7), _CMAX)
        l = jnp.minimum(i - c * LANEB, LANEB - 1)
        _extract_col(buf, l, zb_v, col)

    for idx_v, table, out in ((uidx_v, w_t, zu_out), (vidx_v, h_t, zv_out)):
        for h in range(2):
            e0 = h * _HALF
            for k in range(_DEPTH):
                fire(idx_v, table, bufs[k], sems[k], e0 + k)

            def body(j, _):
                for k in range(_DEPTH):
                    b = e0 + j * _DEPTH + k
                    take(idx_v, bufs[k], sems[k], b, j * _DEPTH + k)
                    bn = jnp.minimum(b + _DEPTH, e0 + _HALF - 1)
                    fire(idx_v, table, bufs[k], sems[k], bn)
                return ()

            lax.fori_loop(0, _HALF // _DEPTH, body, ())
            for k in range(_DEPTH):
                pltpu.make_async_copy(w_t.at[:, pl.ds(0, LANEB)],
                                      bufs[k], sems[k]).wait()
            pltpu.sync_copy(zb_v, out.at[:, pl.ds(base + e0, _HALF)])


def _sc_body_wrapper(xt_hbm, w_t, h_t, zu_out, zv_out,
                     uidx_v, vidx_v,
                     b0, b1, b2, b3, b4, b5, b6, b7, zb_v,
                     s0, s1, s2, s3, s4, s5, s6, s7):
    _sc_gather_body(xt_hbm, w_t, h_t, zu_out, zv_out, uidx_v, vidx_v,
                    (b0, b1, b2, b3, b4, b5, b6, b7), zb_v,
                    (s0, s1, s2, s3, s4, s5, s6, s7))


@jax.jit
def _sc_gather(xt, w_t, h_t):
    mesh = plsc.VectorSubcoreMesh(core_axis_name="c", subcore_axis_name="s")
    f = functools.partial(
        pl.kernel,
        mesh=mesh,
        out_type=[
            jax.ShapeDtypeStruct((EMB_K, CHUNK), jnp.float32),
            jax.ShapeDtypeStruct((EMB_K, CHUNK), jnp.float32),
        ],
        scratch_types=(
            [pltpu.VMEM((_BPW + 16,), jnp.int32)] * 2
            + [pltpu.VMEM((EMB_K, LANEB), jnp.float32)] * _DEPTH
            + [pltpu.VMEM((EMB_K, _HALF), jnp.float32)]
            + [pltpu.SemaphoreType.DMA] * _DEPTH
        ),
        compiler_params=pltpu.CompilerParams(needs_layout_passes=False),
    )(_sc_body_wrapper)
    return f(xt, w_t, h_t)


def _tc_mlp_body(zu_ref, zv_ref, xt_ref, wtu_ref, wtv_ref,
                 a_ref, b_ref, bias_ref, w2_ref, o_ref):
    blk = zu_ref.shape[1]
    kiota = lax.broadcasted_iota(jnp.int32, (EMB_K, blk), 0)

    def fixed(z_ref, idx_row, wt_ref):
        rem = idx_row - _TAIL
        oh = (kiota == rem).astype(jnp.float32)
        repl = lax.dot_general(wt_ref[...], oh, (((1,), (0,)), ((), ())),
                               preferred_element_type=jnp.float32)
        return jnp.where(idx_row >= _TAIL, repl, z_ref[...])

    zu = fixed(zu_ref, xt_ref[0:1, :], wtu_ref)
    zv = fixed(zv_ref, xt_ref[1:2, :], wtv_ref)
    h = lax.dot_general(a_ref[...], zu, (((1,), (0,)), ((), ())),
                        preferred_element_type=jnp.float32)
    h = h + lax.dot_general(b_ref[...], zv, (((1,), (0,)), ((), ())),
                            preferred_element_type=jnp.float32)
    h = h + bias_ref[...]
    h = jnp.maximum(h, 0.0)
    o_ref[...] = lax.dot_general(w2_ref[...], h, (((1,), (0,)), ((), ())),
                                 preferred_element_type=jnp.float32)


_TC_BLK = 2048


@jax.jit
def _tc_mlp(zu, zv, xt, wtu, wtv, a_w, b_w, bias, w2_row):
    grid = (CHUNK // _TC_BLK,)
    return pl.pallas_call(
        _tc_mlp_body,
        grid=grid,
        in_specs=[
            pl.BlockSpec((EMB_K, _TC_BLK), lambda i: (0, i)),
            pl.BlockSpec((EMB_K, _TC_BLK), lambda i: (0, i)),
            pl.BlockSpec((2, _TC_BLK), lambda i: (0, i)),
            pl.BlockSpec((EMB_K, EMB_K), lambda i: (0, 0)),
            pl.BlockSpec((EMB_K, EMB_K), lambda i: (0, 0)),
            pl.BlockSpec((EMB_K, EMB_K), lambda i: (0, 0)),
            pl.BlockSpec((EMB_K, EMB_K), lambda i: (0, 0)),
            pl.BlockSpec((EMB_K, 1), lambda i: (0, 0)),
            pl.BlockSpec((1, EMB_K), lambda i: (0, 0)),
        ],
        out_specs=pl.BlockSpec((1, _TC_BLK), lambda i: (0, i)),
        out_shape=jax.ShapeDtypeStruct((1, CHUNK), jnp.float32),
    )(zu, zv, xt, wtu, wtv, a_w, b_w, bias, w2_row)


def kernel(x, W_table, H_table, linear1_w, linear1_b, linear2_w):
    xt = x.T.astype(jnp.int32)
    w_t = W_table.T
    h_t = H_table.T
    wtu = W_table[_TAIL:, :].T
    wtv = H_table[_TAIL:, :].T
    a_w = linear1_w[:, :EMB_K]
    b_w = linear1_w[:, EMB_K:]
    bias = linear1_b.reshape(EMB_K, 1)
    parts = []
    for n in range(NCHUNK):
        xt_n = lax.slice_in_dim(xt, n * CHUNK, (n + 1) * CHUNK, axis=1)
        zu, zv = _sc_gather(xt_n, w_t, h_t)
        parts.append(_tc_mlp(zu, zv, xt_n, wtu, wtv, a_w, b_w, bias,
                             linear2_w))
    return jnp.concatenate(parts, axis=1).reshape(BATCH, 1)


# speedup vs baseline: 1.0536x; 1.0536x over previous
import functools

import jax
import jax.numpy as jnp
from jax import lax
from jax.experimental import pallas as pl
from jax.experimental.pallas import tpu as pltpu
from jax.experimental.pallas import tpu_sc as plsc

BATCH = 16384
EMB_K = 64
NROWS = 1000000
LANEB = 128
_LAST = (NROWS - 1) // LANEB
_CMAX = _LAST - 1
_TAIL = _LAST * LANEB

_info = plsc.get_sparse_core_info()
_NC, _NS = _info.num_cores, _info.num_subcores
_NW = _NC * _NS
_BPW = BATCH // _NW
_HALF = _BPW // 2
_DEPTH = 8


def _extract_col(zb, l, dst, b):
    lvec = jnp.full((16,), l, jnp.int32)
    bvec = jnp.full((16,), b, jnp.int32)
    for g in range(EMB_K // 16):
        rows = lax.iota(jnp.int32, 16) + (g * 16)
        vals = plsc.load_gather(zb, [rows, lvec])
        plsc.store_scatter(dst, [rows, bvec], vals)


def _sc_gather_body(xt_hbm, w_t, h_t, zu_out, zv_out,
                    uidx_v, vidx_v, bufs, zb_v, sems):
    wid = lax.axis_index("s") * _NC + lax.axis_index("c")
    base = wid * _BPW
    pltpu.sync_copy(xt_hbm.at[0, pl.ds(base, _BPW)], uidx_v.at[pl.ds(0, _BPW)])
    pltpu.sync_copy(xt_hbm.at[1, pl.ds(base, _BPW)], vidx_v.at[pl.ds(0, _BPW)])

    def read_idx(idx_v, b):
        return idx_v[pl.ds(b, 16)][0]

    def fire(idx_v, table, buf, sem, b):
        i = read_idx(idx_v, b)
        c = jnp.minimum(lax.shift_right_logical(i, 7), _CMAX)
        o = pl.multiple_of(c * LANEB, LANEB)
        pltpu.make_async_copy(table.at[:, pl.ds(o, LANEB)], buf, sem).start()

    def take(idx_v, buf, sem, b, col):
        pltpu.make_async_copy(w_t.at[:, pl.ds(0, LANEB)], buf, sem).wait()
        i = read_idx(idx_v, b)
        c = jnp.minimum(lax.shift_right_logical(i, 7), _CMAX)
        l = jnp.minimum(i - c * LANEB, LANEB - 1)
        _extract_col(buf, l, zb_v, col)

    for idx_v, table, out in ((uidx_v, w_t, zu_out), (vidx_v, h_t, zv_out)):
        for h in range(2):
            e0 = h * _HALF
            for k in range(_DEPTH):
                fire(idx_v, table, bufs[k], sems[k], e0 + k)

            def body(j, _):
                for k in range(_DEPTH):
                    b = e0 + j * _DEPTH + k
                    take(idx_v, bufs[k], sems[k], b, j * _DEPTH + k)
                    bn = jnp.minimum(b + _DEPTH, e0 + _HALF - 1)
                    fire(idx_v, table, bufs[k], sems[k], bn)
                return ()

            lax.fori_loop(0, _HALF // _DEPTH, body, ())
            for k in range(_DEPTH):
                pltpu.make_async_copy(w_t.at[:, pl.ds(0, LANEB)],
                                      bufs[k], sems[k]).wait()
            pltpu.sync_copy(zb_v, out.at[:, pl.ds(base + e0, _HALF)])


def _sc_body_wrapper(xt_hbm, w_t, h_t, zu_out, zv_out,
                     uidx_v, vidx_v,
                     b0, b1, b2, b3, b4, b5, b6, b7, zb_v,
                     s0, s1, s2, s3, s4, s5, s6, s7):
    _sc_gather_body(xt_hbm, w_t, h_t, zu_out, zv_out, uidx_v, vidx_v,
                    (b0, b1, b2, b3, b4, b5, b6, b7), zb_v,
                    (s0, s1, s2, s3, s4, s5, s6, s7))


@jax.jit
def _sc_gather(xt, w_t, h_t):
    mesh = plsc.VectorSubcoreMesh(core_axis_name="c", subcore_axis_name="s")
    f = functools.partial(
        pl.kernel,
        mesh=mesh,
        out_type=[
            jax.ShapeDtypeStruct((EMB_K, BATCH), jnp.float32),
            jax.ShapeDtypeStruct((EMB_K, BATCH), jnp.float32),
        ],
        scratch_types=(
            [pltpu.VMEM((_BPW + 16,), jnp.int32)] * 2
            + [pltpu.VMEM((EMB_K, LANEB), jnp.float32)] * _DEPTH
            + [pltpu.VMEM((EMB_K, _HALF), jnp.float32)]
            + [pltpu.SemaphoreType.DMA] * _DEPTH
        ),
        compiler_params=pltpu.CompilerParams(needs_layout_passes=False),
    )(_sc_body_wrapper)
    return f(xt, w_t, h_t)


def _tc_mlp_body(zu_ref, zv_ref, xt_ref, wtu_ref, wtv_ref,
                 a_ref, b_ref, bias_ref, w2_ref, o_ref):
    blk = zu_ref.shape[1]
    kiota = lax.broadcasted_iota(jnp.int32, (EMB_K, blk), 0)

    def fixed(z_ref, idx_row, wt_ref):
        rem = idx_row - _TAIL
        oh = (kiota == rem).astype(jnp.float32)
        repl = lax.dot_general(wt_ref[...], oh, (((1,), (0,)), ((), ())),
                               preferred_element_type=jnp.float32)
        return jnp.where(idx_row >= _TAIL, repl, z_ref[...])

    zu = fixed(zu_ref, xt_ref[0:1, :], wtu_ref)
    zv = fixed(zv_ref, xt_ref[1:2, :], wtv_ref)
    h = lax.dot_general(a_ref[...], zu, (((1,), (0,)), ((), ())),
                        preferred_element_type=jnp.float32)
    h = h + lax.dot_general(b_ref[...], zv, (((1,), (0,)), ((), ())),
                            preferred_element_type=jnp.float32)
    h = h + bias_ref[...]
    h = jnp.maximum(h, 0.0)
    o_ref[...] = lax.dot_general(w2_ref[...], h, (((1,), (0,)), ((), ())),
                                 preferred_element_type=jnp.float32)


_TC_BLK = 2048


@jax.jit
def _tc_mlp(zu, zv, xt, wtu, wtv, a_w, b_w, bias, w2_row):
    grid = (BATCH // _TC_BLK,)
    return pl.pallas_call(
        _tc_mlp_body,
        grid=grid,
        in_specs=[
            pl.BlockSpec((EMB_K, _TC_BLK), lambda i: (0, i)),
            pl.BlockSpec((EMB_K, _TC_BLK), lambda i: (0, i)),
            pl.BlockSpec((2, _TC_BLK), lambda i: (0, i)),
            pl.BlockSpec((EMB_K, EMB_K), lambda i: (0, 0)),
            pl.BlockSpec((EMB_K, EMB_K), lambda i: (0, 0)),
            pl.BlockSpec((EMB_K, EMB_K), lambda i: (0, 0)),
            pl.BlockSpec((EMB_K, EMB_K), lambda i: (0, 0)),
            pl.BlockSpec((EMB_K, 1), lambda i: (0, 0)),
            pl.BlockSpec((1, EMB_K), lambda i: (0, 0)),
        ],
        out_specs=pl.BlockSpec((1, _TC_BLK), lambda i: (0, i)),
        out_shape=jax.ShapeDtypeStruct((1, BATCH), jnp.float32),
    )(zu, zv, xt, wtu, wtv, a_w, b_w, bias, w2_row)


def kernel(x, W_table, H_table, linear1_w, linear1_b, linear2_w):
    xt = x.T.astype(jnp.int32)
    w_t = W_table.T
    h_t = H_table.T
    zu, zv = _sc_gather(xt, w_t, h_t)
    wtu = W_table[_TAIL:, :].T
    wtv = H_table[_TAIL:, :].T
    a_w = linear1_w[:, :EMB_K]
    b_w = linear1_w[:, EMB_K:]
    bias = linear1_b.reshape(EMB_K, 1)
    out_t = _tc_mlp(zu, zv, xt, wtu, wtv, a_w, b_w, bias, linear2_w)
    return out_t.reshape(BATCH, 1)


# speedup vs baseline: 1.0595x; 1.0056x over previous
import functools

import jax
import jax.numpy as jnp
from jax import lax
from jax.experimental import pallas as pl
from jax.experimental.pallas import tpu as pltpu
from jax.experimental.pallas import tpu_sc as plsc

BATCH = 16384
EMB_K = 64
NROWS = 1000000
LANEB = 128
_LAST = (NROWS - 1) // LANEB
_CMAX = _LAST - 1
_TAIL = _LAST * LANEB

_info = plsc.get_sparse_core_info()
_NC, _NS = _info.num_cores, _info.num_subcores
_NW = _NC * _NS
_BPW = BATCH // _NW
_HALF = _BPW // 2
_DEPTH = 8


def _extract_col(zb, l, dst, b):
    lvec = jnp.full((16,), l, jnp.int32)
    bvec = jnp.full((16,), b, jnp.int32)
    for g in range(EMB_K // 16):
        rows = lax.iota(jnp.int32, 16) + (g * 16)
        vals = plsc.load_gather(zb, [rows, lvec])
        plsc.store_scatter(dst, [rows, bvec], vals)


def _sc_gather_body(xt_hbm, w_t, h_t, zu_out, zv_out,
                    uidx_v, vidx_v, bufs, zb_v, sems):
    wid = lax.axis_index("s") * _NC + lax.axis_index("c")
    base = wid * _BPW
    pltpu.sync_copy(xt_hbm.at[0, pl.ds(base, _BPW)], uidx_v.at[pl.ds(0, _BPW)])
    pltpu.sync_copy(xt_hbm.at[1, pl.ds(base, _BPW)], vidx_v.at[pl.ds(0, _BPW)])

    def read_idx(idx_v, b):
        return idx_v[pl.ds(b, 16)][0]

    def fire(idx_v, table, buf, sem, b):
        i = read_idx(idx_v, b)
        c = jnp.minimum(lax.shift_right_logical(i, 7), _CMAX)
        o = pl.multiple_of(c * LANEB, LANEB)
        pltpu.make_async_copy(table.at[:, pl.ds(o, LANEB)], buf, sem).start()

    def take(idx_v, buf, sem, b, col):
        pltpu.make_async_copy(w_t.at[:, pl.ds(0, LANEB)], buf, sem).wait()
        i = read_idx(idx_v, b)
        c = jnp.minimum(lax.shift_right_logical(i, 7), _CMAX)
        l = jnp.minimum(i - c * LANEB, LANEB - 1)
        _extract_col(buf, l, zb_v, col)

    for idx_v, table, out in ((uidx_v, w_t, zu_out), (vidx_v, h_t, zv_out)):
        for h in range(2):
            e0 = h * _HALF
            for k in range(_DEPTH):
                fire(idx_v, table, bufs[k], sems[k], e0 + k)

            def body(j, _):
                for k in range(_DEPTH):
                    b = e0 + j * _DEPTH + k
                    take(idx_v, bufs[k], sems[k], b, j * _DEPTH + k)
                    bn = jnp.minimum(b + _DEPTH, e0 + _HALF - 1)
                    fire(idx_v, table, bufs[k], sems[k], bn)
                return ()

            lax.fori_loop(0, _HALF // _DEPTH, body, ())
            for k in range(_DEPTH):
                pltpu.make_async_copy(w_t.at[:, pl.ds(0, LANEB)],
                                      bufs[k], sems[k]).wait()
            pltpu.sync_copy(zb_v, out.at[:, pl.ds(base + e0, _HALF)])


def _sc_body_wrapper(xt_hbm, w_t, h_t, zu_out, zv_out,
                     uidx_v, vidx_v,
                     b0, b1, b2, b3, b4, b5, b6, b7, zb_v,
                     s0, s1, s2, s3, s4, s5, s6, s7):
    _sc_gather_body(xt_hbm, w_t, h_t, zu_out, zv_out, uidx_v, vidx_v,
                    (b0, b1, b2, b3, b4, b5, b6, b7), zb_v,
                    (s0, s1, s2, s3, s4, s5, s6, s7))


@jax.jit
def _sc_gather(xt, w_t, h_t):
    mesh = plsc.VectorSubcoreMesh(core_axis_name="c", subcore_axis_name="s")
    f = functools.partial(
        pl.kernel,
        mesh=mesh,
        out_type=[
            jax.ShapeDtypeStruct((EMB_K, BATCH), jnp.float32),
            jax.ShapeDtypeStruct((EMB_K, BATCH), jnp.float32),
        ],
        scratch_types=(
            [pltpu.VMEM((_BPW + 16,), jnp.int32)] * 2
            + [pltpu.VMEM((EMB_K, LANEB), jnp.float32)] * _DEPTH
            + [pltpu.VMEM((EMB_K, _HALF), jnp.float32)]
            + [pltpu.SemaphoreType.DMA] * _DEPTH
        ),
        compiler_params=pltpu.CompilerParams(needs_layout_passes=False),
    )(_sc_body_wrapper)
    return f(xt, w_t, h_t)


def _tc_mlp_body(zu_ref, zv_ref, xt_ref, wtu_ref, wtv_ref,
                 a_ref, b_ref, bias_ref, w2_ref, o_ref):
    blk = zu_ref.shape[1]
    kiota = lax.broadcasted_iota(jnp.int32, (EMB_K, blk), 0)

    def fixed(z_ref, idx_row, wt_ref):
        rem = idx_row - _TAIL
        oh = (kiota == rem).astype(jnp.float32)
        repl = lax.dot_general(wt_ref[...], oh, (((1,), (0,)), ((), ())),
                               preferred_element_type=jnp.float32)
        return jnp.where(idx_row >= _TAIL, repl, z_ref[...])

    zu = fixed(zu_ref, xt_ref[0:1, :], wtu_ref)
    zv = fixed(zv_ref, xt_ref[1:2, :], wtv_ref)
    h = lax.dot_general(a_ref[...], zu, (((1,), (0,)), ((), ())),
                        preferred_element_type=jnp.float32)
    h = h + lax.dot_general(b_ref[...], zv, (((1,), (0,)), ((), ())),
                            preferred_element_type=jnp.float32)
    h = h + bias_ref[...]
    h = jnp.maximum(h, 0.0)
    o_ref[...] = lax.dot_general(w2_ref[...], h, (((1,), (0,)), ((), ())),
                                 preferred_element_type=jnp.float32)


_TC_BLK = 8192


@jax.jit
def _tc_mlp(zu, zv, xt, wtu, wtv, a_w, b_w, bias, w2_row):
    grid = (BATCH // _TC_BLK,)
    return pl.pallas_call(
        _tc_mlp_body,
        grid=grid,
        in_specs=[
            pl.BlockSpec((EMB_K, _TC_BLK), lambda i: (0, i)),
            pl.BlockSpec((EMB_K, _TC_BLK), lambda i: (0, i)),
            pl.BlockSpec((2, _TC_BLK), lambda i: (0, i)),
            pl.BlockSpec((EMB_K, EMB_K), lambda i: (0, 0)),
            pl.BlockSpec((EMB_K, EMB_K), lambda i: (0, 0)),
            pl.BlockSpec((EMB_K, EMB_K), lambda i: (0, 0)),
            pl.BlockSpec((EMB_K, EMB_K), lambda i: (0, 0)),
            pl.BlockSpec((EMB_K, 1), lambda i: (0, 0)),
            pl.BlockSpec((1, EMB_K), lambda i: (0, 0)),
        ],
        out_specs=pl.BlockSpec((1, _TC_BLK), lambda i: (0, i)),
        out_shape=jax.ShapeDtypeStruct((1, BATCH), jnp.float32),
    )(zu, zv, xt, wtu, wtv, a_w, b_w, bias, w2_row)


def kernel(x, W_table, H_table, linear1_w, linear1_b, linear2_w):
    xt = x.T.astype(jnp.int32)
    w_t = W_table.T
    h_t = H_table.T
    zu, zv = _sc_gather(xt, w_t, h_t)
    wtu = W_table[_TAIL:, :].T
    wtv = H_table[_TAIL:, :].T
    a_w = linear1_w[:, :EMB_K]
    b_w = linear1_w[:, EMB_K:]
    bias = linear1_b.reshape(EMB_K, 1)
    out_t = _tc_mlp(zu, zv, xt, wtu, wtv, a_w, b_w, bias, linear2_w)
    return out_t.reshape(BATCH, 1)
